# baseline (speedup 1.0000x reference)
import functools

import jax
import jax.numpy as jnp
from jax import lax
from jax.experimental import pallas as pl
from jax.experimental.pallas import tpu as pltpu
from jax.experimental.pallas import tpu_sc as plsc

N = 10000
E = 320000
D = 128
W16 = 16

NC = 2
NS = 16
NW = NC * NS
CHUNK = 128
K = 80
CT = NW * K
EPAD = CT * CHUNK
ROWS = 10240
RPS = ROWS // NS
GARBAGE = N
_NBUF = 4
TPS = N // NS


def _fill(ref, rows, width, value):
    groups = width // 16

    def body(i, _):
        for g in range(groups):
            ref[i, pl.ds(g * 16, 16)] = jnp.full((16,), value, jnp.float32)
        return 0

    lax.fori_loop(0, rows, body, 0)


def _sc_scatter_ones(dst2d):
    mesh = plsc.VectorSubcoreMesh(core_axis_name="c", subcore_axis_name="s")

    @functools.partial(
        pl.kernel, mesh=mesh,
        out_type=jax.ShapeDtypeStruct((NC, ROWS, W16), jnp.float32),
        scratch_types=[
            pltpu.VMEM((K, CHUNK), jnp.int32),
            pltpu.VMEM((CHUNK, W16), jnp.float32),
            pltpu.VMEM((CHUNK, W16), jnp.float32),
            pltpu.VMEM_SHARED((ROWS, W16), jnp.float32),
            pltpu.SemaphoreType.DMA,
        ],
        compiler_params=pltpu.CompilerParams(use_tc_tiling_on_sc=False),
    )
    def k(dst_hbm, out_hbm, dst_v, ones_v, zero_v, acc, sem):
        c = lax.axis_index("c")
        s = lax.axis_index("s")
        wid = s * NC + c
        pltpu.sync_copy(dst_hbm.at[pl.ds(wid * K, K)], dst_v)
        _fill(ones_v, CHUNK, W16, 1.0)
        _fill(zero_v, CHUNK, W16, 0.0)
        for kk in range(RPS // CHUNK):
            pltpu.sync_copy(zero_v, acc.at[pl.ds(s * RPS + kk * CHUNK, CHUNK)])
        plsc.subcore_barrier()

        _W = 16

        def body(j, _):
            pltpu.async_copy(ones_v, acc.at[dst_v.at[j]], sem, add=True)

            @pl.when(j >= _W)
            def _():
                pltpu.make_async_copy(ones_v, acc.at[dst_v.at[0]], sem).wait()
            return 0

        lax.fori_loop(0, K, body, 0)
        for _ in range(_W):
            pltpu.make_async_copy(ones_v, acc.at[dst_v.at[0]], sem).wait()
        plsc.subcore_barrier()
        pltpu.sync_copy(acc.at[pl.ds(s * RPS, RPS)],
                        out_hbm.at[c, pl.ds(s * RPS, RPS)])

    return k(dst2d)


def _sc_gather_scatter(table, src2d, dst2d, width):
    mesh = plsc.VectorSubcoreMesh(core_axis_name="c", subcore_axis_name="s")

    @functools.partial(
        pl.kernel, mesh=mesh,
        out_type=jax.ShapeDtypeStruct((NC, ROWS, width), jnp.float32),
        scratch_types=[
            pltpu.VMEM((K, CHUNK), jnp.int32),
            pltpu.VMEM((K, CHUNK), jnp.int32),
            [pltpu.VMEM((CHUNK, width), jnp.float32) for _ in range(_NBUF)],
            pltpu.VMEM_SHARED((N, width), jnp.float32),
            pltpu.VMEM_SHARED((ROWS, width), jnp.float32),
            [pltpu.SemaphoreType.DMA for _ in range(_NBUF)],
            [pltpu.SemaphoreType.DMA for _ in range(_NBUF)],
        ],
        compiler_params=pltpu.CompilerParams(use_tc_tiling_on_sc=False),
    )
    def k(tab_hbm, src_hbm, dst_hbm, out_hbm,
          src_v, dst_v, rows, tab_s, acc, gsems, ssems):
        c = lax.axis_index("c")
        s = lax.axis_index("s")
        wid = s * NC + c
        pltpu.sync_copy(tab_hbm.at[pl.ds(s * TPS, TPS)],
                        tab_s.at[pl.ds(s * TPS, TPS)])
        pltpu.sync_copy(src_hbm.at[pl.ds(wid * K, K)], src_v)
        pltpu.sync_copy(dst_hbm.at[pl.ds(wid * K, K)], dst_v)
        _fill(rows[0], CHUNK, width, 0.0)
        for kk in range(RPS // CHUNK):
            pltpu.sync_copy(rows[0], acc.at[pl.ds(s * RPS + kk * CHUNK, CHUNK)])
        plsc.subcore_barrier()

        def wrap(j):
            return jnp.where(j >= K, j - K, j)

        for b in range(_NBUF):
            pltpu.async_copy(tab_s.at[src_v.at[b]], rows[b], gsems[b])

        def body(jj, _):
            j = _NBUF * jj
            for b in range(_NBUF):
                pltpu.make_async_copy(
                    tab_s.at[src_v.at[j + b]], rows[b], gsems[b]).wait()
                pltpu.sync_copy(rows[b], acc.at[dst_v.at[j + b]], add=True)
                pltpu.async_copy(
                    tab_s.at[src_v.at[wrap(j + b + _NBUF)]], rows[b], gsems[b])
            return 0

        lax.fori_loop(0, K // _NBUF, body, 0)
        for b in range(_NBUF):
            pltpu.make_async_copy(
                tab_s.at[src_v.at[b]], rows[b], gsems[b]).wait()
        plsc.subcore_barrier()
        pltpu.sync_copy(acc.at[pl.ds(s * RPS, RPS)],
                        out_hbm.at[c, pl.ds(s * RPS, RPS)])

    return k(table, src2d, dst2d)


_NRELOAD = 4
_KH = CT // NS // _NRELOAD


def _sc_layer1(y2, src2d, dst2d):
    width = D // 2
    mesh = plsc.VectorSubcoreMesh(core_axis_name="c", subcore_axis_name="s")

    @functools.partial(
        pl.kernel, mesh=mesh,
        out_type=jax.ShapeDtypeStruct((NC, ROWS, width), jnp.float32),
        scratch_types=[
            pltpu.VMEM((_KH, CHUNK), jnp.int32),
            pltpu.VMEM((_KH, CHUNK), jnp.int32),
            [pltpu.VMEM((CHUNK, width), jnp.float32) for _ in range(_NBUF)],
            pltpu.VMEM_SHARED((N, width), jnp.float32),
            pltpu.VMEM_SHARED((ROWS, width), jnp.float32),
            [pltpu.SemaphoreType.DMA for _ in range(_NBUF)],
            [pltpu.SemaphoreType.DMA for _ in range(_NBUF)],
        ],
        compiler_params=pltpu.CompilerParams(use_tc_tiling_on_sc=False),
    )
    def k(y2_hbm, src_hbm, dst_hbm, out_hbm,
          src_v, dst_v, rows, tab_s, acc, gsems, ssems):
        c = lax.axis_index("c")
        s = lax.axis_index("s")
        pltpu.sync_copy(y2_hbm.at[c, pl.ds(s * TPS, TPS)],
                        tab_s.at[pl.ds(s * TPS, TPS)])
        _fill(rows[0], CHUNK, width, 0.0)
        for kk in range(RPS // CHUNK):
            pltpu.sync_copy(rows[0], acc.at[pl.ds(s * RPS + kk * CHUNK, CHUNK)])
        plsc.subcore_barrier()

        def wrap(j):
            return jnp.where(j >= _KH, j - _KH, j)

        for half in range(_NRELOAD):
            base = s * _NRELOAD * _KH + half * _KH
            pltpu.sync_copy(src_hbm.at[pl.ds(base, _KH)], src_v)
            pltpu.sync_copy(dst_hbm.at[pl.ds(base, _KH)], dst_v)
            for b in range(_NBUF):
                pltpu.async_copy(tab_s.at[src_v.at[b]], rows[b], gsems[b])

            def body(jj, _):
                j = _NBUF * jj
                for b in range(_NBUF):
                    pltpu.make_async_copy(
                        tab_s.at[src_v.at[j + b]], rows[b], gsems[b]).wait()
                    pltpu.sync_copy(rows[b], acc.at[dst_v.at[j + b]], add=True)
                    pltpu.async_copy(
                        tab_s.at[src_v.at[wrap(j + b + _NBUF)]],
                        rows[b], gsems[b])
                return 0

            lax.fori_loop(0, _KH // _NBUF, body, 0)
            for b in range(_NBUF):
                pltpu.make_async_copy(
                    tab_s.at[src_v.at[b]], rows[b], gsems[b]).wait()

        plsc.subcore_barrier()
        pltpu.sync_copy(acc.at[pl.ds(s * RPS, RPS)],
                        out_hbm.at[c, pl.ds(s * RPS, RPS)])

    return k(y2, src2d, dst2d)


_BLK = 1000


def _dis_from(dego_ref):
    d0 = dego_ref[0, :, 0:1]
    d1 = dego_ref[1, :, 0:1]
    return lax.rsqrt(1.0 + d0 + d1)


def _tc_matmul_scale(x, W, dego):
    def body(x_ref, w_ref, dego_ref, o_ref):
        xw = jnp.dot(x_ref[...], w_ref[...],
                     preferred_element_type=jnp.float32)
        y = xw * _dis_from(dego_ref)
        o_ref[0] = y[:, :D // 2]
        o_ref[1] = y[:, D // 2:]

    return pl.pallas_call(
        body,
        grid=(N // _BLK,),
        in_specs=[pl.BlockSpec((_BLK, D), lambda i: (i, 0)),
                  pl.BlockSpec((D, D), lambda i: (0, 0)),
                  pl.BlockSpec((2, _BLK, W16), lambda i: (0, i, 0))],
        out_specs=pl.BlockSpec((2, _BLK, D // 2), lambda i: (0, i, 0)),
        out_shape=jax.ShapeDtypeStruct((2, N, D // 2), jnp.float32),
    )(x, W, dego)


def _tc_layer1_finish(acc1, y2, dego, b1, W2p):
    def body(a_ref, y_ref, dego_ref, b1_ref, w2_ref, o_ref):
        dis = _dis_from(dego_ref)
        s0 = a_ref[0] + y_ref[0]
        s1 = a_ref[1] + y_ref[1]
        ssum = jnp.concatenate([s0, s1], axis=1)
        h = jnp.maximum(ssum * dis + b1_ref[...][None, :], 0.0)
        o_ref[...] = jnp.dot(h, w2_ref[...],
                             preferred_element_type=jnp.float32) * dis

    return pl.pallas_call(
        body,
        grid=(N // _BLK,),
        in_specs=[pl.BlockSpec((2, _BLK, D // 2), lambda i: (0, i, 0)),
                  pl.BlockSpec((2, _BLK, D // 2), lambda i: (0, i, 0)),
                  pl.BlockSpec((2, _BLK, W16), lambda i: (0, i, 0)),
                  pl.BlockSpec((D,), lambda i: (0,)),
                  pl.BlockSpec((D, W16), lambda i: (0, 0))],
        out_specs=pl.BlockSpec((_BLK, W16), lambda i: (i, 0)),
        out_shape=jax.ShapeDtypeStruct((N, W16), jnp.float32),
    )(acc1, y2, dego, b1, W2p)


def _tc_layer2_finish(acc2, z, dego, b2p):
    def body(a_ref, z_ref, dego_ref, b2_ref, o_ref):
        dis = _dis_from(dego_ref)
        ssum = a_ref[0] + a_ref[1] + z_ref[...]
        o_ref[...] = ssum * dis + b2_ref[...][None, :]

    return pl.pallas_call(
        body,
        grid=(N // _BLK,),
        in_specs=[pl.BlockSpec((2, _BLK, W16), lambda i: (0, i, 0)),
                  pl.BlockSpec((_BLK, W16), lambda i: (i, 0)),
                  pl.BlockSpec((2, _BLK, W16), lambda i: (0, i, 0)),
                  pl.BlockSpec((W16,), lambda i: (0,))],
        out_specs=pl.BlockSpec((_BLK, W16), lambda i: (i, 0)),
        out_shape=jax.ShapeDtypeStruct((N, W16), jnp.float32),
    )(acc2, z, dego, b2p)


def kernel(x, edge_index, W1, b1, W2, b2):
    src = edge_index[0].astype(jnp.int32)
    dst = edge_index[1].astype(jnp.int32)
    pad = EPAD - E
    src2d = jnp.concatenate(
        [src, jnp.zeros((pad,), jnp.int32)]).reshape(CT, CHUNK)
    dst2d = jnp.concatenate(
        [dst, jnp.full((pad,), GARBAGE, jnp.int32)]).reshape(CT, CHUNK)
    W2p = jnp.pad(W2, ((0, 0), (0, W16 - W2.shape[1])))
    b2p = jnp.pad(b2, (0, W16 - b2.shape[0]))

    dego = _sc_scatter_ones(dst2d)
    y2 = _tc_matmul_scale(x, W1, dego)
    acc1 = _sc_layer1(y2, src2d, dst2d)
    z = _tc_layer1_finish(acc1, y2, dego, b1, W2p)
    acc2 = _sc_gather_scatter(z, src2d, dst2d, W16)
    out = _tc_layer2_finish(acc2, z, dego, b2p)
    return out[:, :2]


# speedup vs baseline: 1.0238x; 1.0238x over previous
import functools

import jax
import jax.numpy as jnp
from jax import lax
from jax.experimental import pallas as pl
from jax.experimental.pallas import tpu as pltpu
from jax.experimental.pallas import tpu_sc as plsc

N = 10000
E = 320000
D = 128
W16 = 16

NC = 2
NS = 16
NW = NC * NS
CHUNK = 128
K = 80
CT = NW * K
EPAD = CT * CHUNK
ROWS = 10240
RPS = ROWS // NS
GARBAGE = N
_NBUF = 2
TPS = N // NS


def _fill(ref, rows, width, value):
    groups = width // 16

    def body(i, _):
        for g in range(groups):
            ref[i, pl.ds(g * 16, 16)] = jnp.full((16,), value, jnp.float32)
        return 0

    lax.fori_loop(0, rows, body, 0)


def _sc_scatter_ones(dst2d):
    mesh = plsc.VectorSubcoreMesh(core_axis_name="c", subcore_axis_name="s")

    @functools.partial(
        pl.kernel, mesh=mesh,
        out_type=jax.ShapeDtypeStruct((NC, ROWS, W16), jnp.float32),
        scratch_types=[
            pltpu.VMEM((K, CHUNK), jnp.int32),
            pltpu.VMEM((CHUNK, W16), jnp.float32),
            pltpu.VMEM((CHUNK, W16), jnp.float32),
            pltpu.VMEM_SHARED((ROWS, W16), jnp.float32),
            pltpu.SemaphoreType.DMA,
        ],
        compiler_params=pltpu.CompilerParams(use_tc_tiling_on_sc=False),
    )
    def k(dst_hbm, out_hbm, dst_v, ones_v, zero_v, acc, sem):
        c = lax.axis_index("c")
        s = lax.axis_index("s")
        wid = s * NC + c
        pltpu.sync_copy(dst_hbm.at[pl.ds(wid * K, K)], dst_v)
        _fill(ones_v, CHUNK, W16, 1.0)
        _fill(zero_v, CHUNK, W16, 0.0)
        for kk in range(RPS // CHUNK):
            pltpu.sync_copy(zero_v, acc.at[pl.ds(s * RPS + kk * CHUNK, CHUNK)])
        plsc.subcore_barrier()

        _W = 16

        def body(j, _):
            pltpu.async_copy(ones_v, acc.at[dst_v.at[j]], sem, add=True)

            @pl.when(j >= _W)
            def _():
                pltpu.make_async_copy(ones_v, acc.at[dst_v.at[0]], sem).wait()
            return 0

        lax.fori_loop(0, K, body, 0)
        for _ in range(_W):
            pltpu.make_async_copy(ones_v, acc.at[dst_v.at[0]], sem).wait()
        plsc.subcore_barrier()
        pltpu.sync_copy(acc.at[pl.ds(s * RPS, RPS)],
                        out_hbm.at[c, pl.ds(s * RPS, RPS)])

    return k(dst2d)


def _sc_gather_scatter(table, src2d, dst2d, width):
    mesh = plsc.VectorSubcoreMesh(core_axis_name="c", subcore_axis_name="s")

    @functools.partial(
        pl.kernel, mesh=mesh,
        out_type=jax.ShapeDtypeStruct((NC, ROWS, width), jnp.float32),
        scratch_types=[
            pltpu.VMEM((K, CHUNK), jnp.int32),
            pltpu.VMEM((K, CHUNK), jnp.int32),
            [pltpu.VMEM((CHUNK, width), jnp.float32) for _ in range(_NBUF)],
            pltpu.VMEM_SHARED((N, width), jnp.float32),
            pltpu.VMEM_SHARED((ROWS, width), jnp.float32),
            [pltpu.SemaphoreType.DMA for _ in range(_NBUF)],
            [pltpu.SemaphoreType.DMA for _ in range(_NBUF)],
        ],
        compiler_params=pltpu.CompilerParams(use_tc_tiling_on_sc=False),
    )
    def k(tab_hbm, src_hbm, dst_hbm, out_hbm,
          src_v, dst_v, rows, tab_s, acc, gsems, ssems):
        c = lax.axis_index("c")
        s = lax.axis_index("s")
        wid = s * NC + c
        pltpu.sync_copy(tab_hbm.at[pl.ds(s * TPS, TPS)],
                        tab_s.at[pl.ds(s * TPS, TPS)])
        pltpu.sync_copy(src_hbm.at[pl.ds(wid * K, K)], src_v)
        pltpu.sync_copy(dst_hbm.at[pl.ds(wid * K, K)], dst_v)

        @pl.when(c == 0)
        def _():
            pltpu.sync_copy(tab_hbm.at[pl.ds(s * RPS, RPS)],
                            acc.at[pl.ds(s * RPS, RPS)])

        @pl.when(c != 0)
        def _():
            _fill(rows[0], CHUNK, width, 0.0)
            for kk in range(RPS // CHUNK):
                pltpu.sync_copy(rows[0],
                                acc.at[pl.ds(s * RPS + kk * CHUNK, CHUNK)])
        plsc.subcore_barrier()

        def wrap(j):
            return jnp.where(j >= K, j - K, j)

        for b in range(_NBUF):
            pltpu.async_copy(tab_s.at[src_v.at[b]], rows[b], gsems[b])

        def body(jj, _):
            j = _NBUF * jj
            for b in range(_NBUF):
                pltpu.make_async_copy(
                    tab_s.at[src_v.at[j + b]], rows[b], gsems[b]).wait()
                pltpu.sync_copy(rows[b], acc.at[dst_v.at[j + b]], add=True)
                pltpu.async_copy(
                    tab_s.at[src_v.at[wrap(j + b + _NBUF)]], rows[b], gsems[b])
            return 0

        lax.fori_loop(0, K // _NBUF, body, 0)
        for b in range(_NBUF):
            pltpu.make_async_copy(
                tab_s.at[src_v.at[b]], rows[b], gsems[b]).wait()
        plsc.subcore_barrier()
        pltpu.sync_copy(acc.at[pl.ds(s * RPS, RPS)],
                        out_hbm.at[c, pl.ds(s * RPS, RPS)])

    return k(table, src2d, dst2d)


_NRELOAD = 2
_KH = CT // NS // _NRELOAD


def _sc_layer1(y2, src2d, dst2d):
    width = D // 2
    mesh = plsc.VectorSubcoreMesh(core_axis_name="c", subcore_axis_name="s")

    @functools.partial(
        pl.kernel, mesh=mesh,
        out_type=jax.ShapeDtypeStruct((NC, ROWS, width), jnp.float32),
        scratch_types=[
            pltpu.VMEM((_KH, CHUNK), jnp.int32),
            pltpu.VMEM((_KH, CHUNK), jnp.int32),
            [pltpu.VMEM((CHUNK, width), jnp.float32) for _ in range(_NBUF)],
            pltpu.VMEM_SHARED((N, width), jnp.float32),
            pltpu.VMEM_SHARED((ROWS, width), jnp.float32),
            [pltpu.SemaphoreType.DMA for _ in range(_NBUF)],
            [pltpu.SemaphoreType.DMA for _ in range(_NBUF)],
        ],
        compiler_params=pltpu.CompilerParams(use_tc_tiling_on_sc=False),
    )
    def k(y2_hbm, src_hbm, dst_hbm, out_hbm,
          src_v, dst_v, rows, tab_s, acc, gsems, ssems):
        c = lax.axis_index("c")
        s = lax.axis_index("s")
        pltpu.sync_copy(y2_hbm.at[c, pl.ds(s * TPS, TPS)],
                        tab_s.at[pl.ds(s * TPS, TPS)])
        pltpu.sync_copy(y2_hbm.at[c, pl.ds(s * RPS, RPS)],
                        acc.at[pl.ds(s * RPS, RPS)])
        plsc.subcore_barrier()

        def wrap(j):
            return jnp.where(j >= _KH, j - _KH, j)

        for half in range(_NRELOAD):
            base = s * _NRELOAD * _KH + half * _KH
            pltpu.sync_copy(src_hbm.at[pl.ds(base, _KH)], src_v)
            pltpu.sync_copy(dst_hbm.at[pl.ds(base, _KH)], dst_v)
            for b in range(_NBUF):
                pltpu.async_copy(tab_s.at[src_v.at[b]], rows[b], gsems[b])

            def body(jj, _):
                j = _NBUF * jj
                for b in range(_NBUF):
                    pltpu.make_async_copy(
                        tab_s.at[src_v.at[j + b]], rows[b], gsems[b]).wait()
                    pltpu.sync_copy(rows[b], acc.at[dst_v.at[j + b]], add=True)
                    pltpu.async_copy(
                        tab_s.at[src_v.at[wrap(j + b + _NBUF)]],
                        rows[b], gsems[b])
                return 0

            lax.fori_loop(0, _KH // _NBUF, body, 0)
            for b in range(_NBUF):
                pltpu.make_async_copy(
                    tab_s.at[src_v.at[b]], rows[b], gsems[b]).wait()

        plsc.subcore_barrier()
        pltpu.sync_copy(acc.at[pl.ds(s * RPS, RPS)],
                        out_hbm.at[c, pl.ds(s * RPS, RPS)])

    return k(y2, src2d, dst2d)


_BLK = 1000


def _dis_from(dego_ref):
    d0 = dego_ref[0, :, 0:1]
    d1 = dego_ref[1, :, 0:1]
    return lax.rsqrt(1.0 + d0 + d1)


def _tc_matmul_scale(x, W, dego):
    def body(x_ref, w_ref, dego_ref, o_ref):
        xw = jnp.dot(x_ref[...], w_ref[...],
                     preferred_element_type=jnp.float32)
        y = xw * _dis_from(dego_ref)
        o_ref[0] = y[:, :D // 2]
        o_ref[1] = y[:, D // 2:]

    return pl.pallas_call(
        body,
        grid=(N // _BLK,),
        in_specs=[pl.BlockSpec((_BLK, D), lambda i: (i, 0)),
                  pl.BlockSpec((D, D), lambda i: (0, 0)),
                  pl.BlockSpec((2, _BLK, W16), lambda i: (0, i, 0))],
        out_specs=pl.BlockSpec((2, _BLK, D // 2), lambda i: (0, i, 0)),
        out_shape=jax.ShapeDtypeStruct((2, ROWS, D // 2), jnp.float32),
    )(x, W, dego)


def _tc_layer1_finish(acc1, dego, b1, W2p):
    def body(a_ref, dego_ref, b1_ref, w2_ref, o_ref):
        dis = _dis_from(dego_ref)
        ssum = jnp.concatenate([a_ref[0], a_ref[1]], axis=1)
        h = jnp.maximum(ssum * dis + b1_ref[...][None, :], 0.0)
        o_ref[...] = jnp.dot(h, w2_ref[...],
                             preferred_element_type=jnp.float32) * dis

    return pl.pallas_call(
        body,
        grid=(N // _BLK,),
        in_specs=[pl.BlockSpec((2, _BLK, D // 2), lambda i: (0, i, 0)),
                  pl.BlockSpec((2, _BLK, W16), lambda i: (0, i, 0)),
                  pl.BlockSpec((D,), lambda i: (0,)),
                  pl.BlockSpec((D, W16), lambda i: (0, 0))],
        out_specs=pl.BlockSpec((_BLK, W16), lambda i: (i, 0)),
        out_shape=jax.ShapeDtypeStruct((ROWS, W16), jnp.float32),
    )(acc1, dego, b1, W2p)


def _tc_layer2_finish(acc2, dego, b2p):
    def body(a_ref, dego_ref, b2_ref, o_ref):
        dis = _dis_from(dego_ref)
        ssum = a_ref[0] + a_ref[1]
        o_ref[...] = ssum * dis + b2_ref[...][None, :]

    return pl.pallas_call(
        body,
        grid=(N // _BLK,),
        in_specs=[pl.BlockSpec((2, _BLK, W16), lambda i: (0, i, 0)),
                  pl.BlockSpec((2, _BLK, W16), lambda i: (0, i, 0)),
                  pl.BlockSpec((W16,), lambda i: (0,))],
        out_specs=pl.BlockSpec((_BLK, W16), lambda i: (i, 0)),
        out_shape=jax.ShapeDtypeStruct((N, W16), jnp.float32),
    )(acc2, dego, b2p)


def kernel(x, edge_index, W1, b1, W2, b2):
    src = edge_index[0].astype(jnp.int32)
    dst = edge_index[1].astype(jnp.int32)
    pad = EPAD - E
    src2d = jnp.concatenate(
        [src, jnp.zeros((pad,), jnp.int32)]).reshape(CT, CHUNK)
    dst2d = jnp.concatenate(
        [dst, jnp.full((pad,), GARBAGE, jnp.int32)]).reshape(CT, CHUNK)
    W2p = jnp.pad(W2, ((0, 0), (0, W16 - W2.shape[1])))
    b2p = jnp.pad(b2, (0, W16 - b2.shape[0]))

    dego = _sc_scatter_ones(dst2d)
    y2 = _tc_matmul_scale(x, W1, dego)
    acc1 = _sc_layer1(y2, src2d, dst2d)
    z = _tc_layer1_finish(acc1, dego, b1, W2p)
    acc2 = _sc_gather_scatter(z, src2d, dst2d, W16)
    out = _tc_layer2_finish(acc2, dego, b2p)
    return out[:, :2]
